# baseline (device time: 33210 ns/iter reference)
import jax
import jax.numpy as jnp
from jax import lax
from jax.experimental import pallas as pl
from jax.experimental.pallas import tpu as pltpu

N_DEV = 4
M = 1024
N = 1024
H = M // 2
Q = M // 4
C = N // 2

F32 = jnp.float32
BF16 = jnp.bfloat16


def _gelu(z):
    return 0.5 * z * (1.0 + jnp.tanh(0.7978845608 * (z + 0.044715 * z * z * z)))


def kernel(A, B):
    def body(
        a_ref,
        b_ref,
        out_ref,
        h_send,
        h_recv,
        q_send,
        q_recv,
        gh_send,
        gh_recv,
        send_sems,
        recv_sems,
    ):
        d = lax.axis_index("i")
        p1 = d ^ 1
        p2 = 3 - d

        keep0 = (d ^ (d >> 1)) & 1
        qi0 = keep0 * 2 + (d >> 1)
        qo0 = keep0 * 2 + (1 - (d >> 1))
        keep1 = d >> 1
        qi1 = keep1 * 2 + (d & 1)
        qo1 = keep1 * 2 + (1 - (d & 1))

        groups = [
            dict(g=0, pa=p1, pb=p2, keep=keep0, qi=qi0, qo=qo0, col=0),
            dict(g=1, pa=p2, pb=p1, keep=keep1, qi=qi1, qo=qo1, col=C),
        ]
        for gr in groups:
            gr["keep_r"] = gr["keep"] * H
            gr["send_r"] = (1 - gr["keep"]) * H
            gr["qi_r"] = gr["qi"] * Q
            gr["qo_r"] = gr["qo"] * Q
            gr["off_qi"] = gr["qi_r"] - gr["keep_r"]
            gr["off_qo"] = gr["qo_r"] - gr["keep_r"]

        barrier_sem = pltpu.get_barrier_semaphore()
        for nbr in [p1, p2]:
            pl.semaphore_signal(
                barrier_sem,
                inc=1,
                device_id=(nbr,),
                device_id_type=pl.DeviceIdType.MESH,
            )
        pl.semaphore_wait(barrier_sem, 2)

        def mm(r, nrows, c):
            a = a_ref[pl.ds(r, nrows), :].astype(BF16)
            b = b_ref[:, pl.ds(c, C)].astype(BF16)
            return jnp.dot(a, b, preferred_element_type=F32)

        rdma1 = []
        for gr in groups:
            h_send[gr["g"]] = mm(gr["send_r"], H, gr["col"]).astype(BF16)
            r = pltpu.make_async_remote_copy(
                src_ref=h_send.at[gr["g"]],
                dst_ref=h_recv.at[gr["g"]],
                send_sem=send_sems.at[gr["g"], 0],
                recv_sem=recv_sems.at[gr["g"], 0],
                device_id=(gr["pa"],),
                device_id_type=pl.DeviceIdType.MESH,
            )
            r.start()
            rdma1.append(r)
        mm_qo = [mm(gr["qo_r"], Q, gr["col"]) for gr in groups]
        mm_qi = [mm(gr["qi_r"], Q, gr["col"]) for gr in groups]

        rdma2 = []
        for gr, r1 in zip(groups, rdma1):
            r1.wait()
            g = gr["g"]
            q_send[g] = (
                mm_qo[g] + h_recv[g, pl.ds(gr["off_qo"], Q), :].astype(F32)
            ).astype(BF16)
            r = pltpu.make_async_remote_copy(
                src_ref=q_send.at[g],
                dst_ref=q_recv.at[g],
                send_sem=send_sems.at[g, 1],
                recv_sem=recv_sems.at[g, 1],
                device_id=(gr["pb"],),
                device_id_type=pl.DeviceIdType.MESH,
            )
            r.start()
            rdma2.append(r)
        zqp = [
            mm_qi[gr["g"]]
            + h_recv[gr["g"], pl.ds(gr["off_qi"], Q), :].astype(F32)
            for gr in groups
        ]

        rdma3 = []
        for gr, r2 in zip(groups, rdma2):
            r2.wait()
            g = gr["g"]
            gq = _gelu(zqp[g] + q_recv[g].astype(F32))
            gh_send[g, pl.ds(gr["off_qi"], Q), :] = gq.astype(BF16)
            r = pltpu.make_async_remote_copy(
                src_ref=gh_send.at[g, pl.ds(gr["off_qi"], Q), :],
                dst_ref=gh_send.at[g, pl.ds(gr["off_qi"], Q), :],
                send_sem=send_sems.at[g, 2],
                recv_sem=recv_sems.at[g, 2],
                device_id=(gr["pb"],),
                device_id_type=pl.DeviceIdType.MESH,
            )
            r.start()
            rdma3.append(r)
            out_ref[pl.ds(gr["qi_r"], Q), pl.ds(gr["col"], C)] = gq

        rdma4 = []
        for gr, r3 in zip(groups, rdma3):
            r3.wait()
            g = gr["g"]
            r = pltpu.make_async_remote_copy(
                src_ref=gh_send.at[g],
                dst_ref=gh_recv.at[g],
                send_sem=send_sems.at[g, 3],
                recv_sem=recv_sems.at[g, 3],
                device_id=(gr["pa"],),
                device_id_type=pl.DeviceIdType.MESH,
            )
            r.start()
            rdma4.append(r)
            out_ref[pl.ds(gr["qo_r"], Q), pl.ds(gr["col"], C)] = gh_send[
                g, pl.ds(gr["off_qo"], Q), :
            ].astype(F32)
        for gr, r4 in zip(groups, rdma4):
            r4.wait()
            out_ref[pl.ds(gr["send_r"], H), pl.ds(gr["col"], C)] = gh_recv[
                gr["g"]
            ].astype(F32)

    return pl.pallas_call(
        body,
        out_shape=jax.ShapeDtypeStruct((M, N), F32),
        in_specs=[
            pl.BlockSpec(memory_space=pltpu.VMEM),
            pl.BlockSpec(memory_space=pltpu.VMEM),
        ],
        out_specs=pl.BlockSpec(memory_space=pltpu.VMEM),
        scratch_shapes=[
            pltpu.VMEM((2, H, C), BF16),
            pltpu.VMEM((2, H, C), BF16),
            pltpu.VMEM((2, Q, C), BF16),
            pltpu.VMEM((2, Q, C), BF16),
            pltpu.VMEM((2, H, C), BF16),
            pltpu.VMEM((2, H, C), BF16),
            pltpu.SemaphoreType.DMA((2, 4)),
            pltpu.SemaphoreType.DMA((2, 4)),
        ],
        compiler_params=pltpu.CompilerParams(collective_id=0),
    )(A, B)


# device time: 32557 ns/iter; 1.0201x vs baseline; 1.0201x over previous
import jax
import jax.numpy as jnp
from jax import lax
from jax.experimental import pallas as pl
from jax.experimental.pallas import tpu as pltpu

N_DEV = 4
M = 1024
N = 1024
H = M // 2
Q = M // 4
C = N // 2

F32 = jnp.float32
BF16 = jnp.bfloat16


def _gelu(z):
    return 0.5 * z * (1.0 + jnp.tanh(0.7978845608 * (z + 0.044715 * z * z * z)))


def kernel(A, B):
    def body(
        a_ref,
        b_ref,
        out_ref,
        h_send,
        h_recv,
        q_send,
        q_recv,
        send_sems,
        recv_sems,
    ):
        d = lax.axis_index("i")
        p1 = d ^ 1
        p2 = 3 - d

        keep0 = (d ^ (d >> 1)) & 1
        qi0 = keep0 * 2 + (d >> 1)
        qo0 = keep0 * 2 + (1 - (d >> 1))
        keep1 = d >> 1
        qi1 = keep1 * 2 + (d & 1)
        qo1 = keep1 * 2 + (1 - (d & 1))

        groups = [
            dict(g=0, pa=p1, pb=p2, keep=keep0, qi=qi0, qo=qo0, col=0),
            dict(g=1, pa=p2, pb=p1, keep=keep1, qi=qi1, qo=qo1, col=C),
        ]
        for gr in groups:
            gr["keep_r"] = gr["keep"] * H
            gr["send_r"] = (1 - gr["keep"]) * H
            gr["qi_r"] = gr["qi"] * Q
            gr["qo_r"] = gr["qo"] * Q
            gr["off_qi"] = gr["qi_r"] - gr["keep_r"]
            gr["off_qo"] = gr["qo_r"] - gr["keep_r"]

        barrier_sem = pltpu.get_barrier_semaphore()
        for nbr in [p1, p2]:
            pl.semaphore_signal(
                barrier_sem,
                inc=1,
                device_id=(nbr,),
                device_id_type=pl.DeviceIdType.MESH,
            )
        pl.semaphore_wait(barrier_sem, 2)

        def mm(r, nrows, c):
            a = a_ref[pl.ds(r, nrows), :].astype(BF16)
            b = b_ref[:, pl.ds(c, C)].astype(BF16)
            return jnp.dot(a, b, preferred_element_type=F32)

        rdma1 = []
        for gr in groups:
            h_send[gr["g"]] = mm(gr["send_r"], H, gr["col"]).astype(BF16)
            r = pltpu.make_async_remote_copy(
                src_ref=h_send.at[gr["g"]],
                dst_ref=h_recv.at[gr["g"]],
                send_sem=send_sems.at[gr["g"], 0],
                recv_sem=recv_sems.at[gr["g"], 0],
                device_id=(gr["pa"],),
                device_id_type=pl.DeviceIdType.MESH,
            )
            r.start()
            rdma1.append(r)
        mm_qo = [mm(gr["qo_r"], Q, gr["col"]) for gr in groups]
        mm_qi = [mm(gr["qi_r"], Q, gr["col"]) for gr in groups]

        rdma2 = []
        for gr, r1 in zip(groups, rdma1):
            r1.wait()
            g = gr["g"]
            q_send[g] = (
                mm_qo[g] + h_recv[g, pl.ds(gr["off_qo"], Q), :].astype(F32)
            ).astype(BF16)
            r = pltpu.make_async_remote_copy(
                src_ref=q_send.at[g],
                dst_ref=q_recv.at[g],
                send_sem=send_sems.at[g, 1],
                recv_sem=recv_sems.at[g, 1],
                device_id=(gr["pb"],),
                device_id_type=pl.DeviceIdType.MESH,
            )
            r.start()
            rdma2.append(r)
        zqp = [
            mm_qi[gr["g"]]
            + h_recv[gr["g"], pl.ds(gr["off_qi"], Q), :].astype(F32)
            for gr in groups
        ]

        rdma3 = []
        for gr, r2 in zip(groups, rdma2):
            r2.wait()
            g = gr["g"]
            gq = _gelu(zqp[g] + q_recv[g].astype(F32))
            qs = (pl.ds(gr["qi_r"], Q), pl.ds(gr["col"], C))
            out_ref[qs] = gq.astype(BF16)
            r = pltpu.make_async_remote_copy(
                src_ref=out_ref.at[qs],
                dst_ref=out_ref.at[qs],
                send_sem=send_sems.at[g, 2],
                recv_sem=recv_sems.at[g, 2],
                device_id=(gr["pb"],),
                device_id_type=pl.DeviceIdType.MESH,
            )
            r.start()
            rdma3.append(r)

        rdma4 = []
        for gr, r3 in zip(groups, rdma3):
            r3.wait()
            g = gr["g"]
            hs = (pl.ds(gr["keep_r"], H), pl.ds(gr["col"], C))
            r = pltpu.make_async_remote_copy(
                src_ref=out_ref.at[hs],
                dst_ref=out_ref.at[hs],
                send_sem=send_sems.at[g, 3],
                recv_sem=recv_sems.at[g, 3],
                device_id=(gr["pa"],),
                device_id_type=pl.DeviceIdType.MESH,
            )
            r.start()
            rdma4.append(r)
        for r4 in rdma4:
            r4.wait()

    return pl.pallas_call(
        body,
        out_shape=jax.ShapeDtypeStruct((M, N), BF16),
        in_specs=[
            pl.BlockSpec(memory_space=pltpu.VMEM),
            pl.BlockSpec(memory_space=pltpu.VMEM),
        ],
        out_specs=pl.BlockSpec(memory_space=pltpu.VMEM),
        scratch_shapes=[
            pltpu.VMEM((2, H, C), BF16),
            pltpu.VMEM((2, H, C), BF16),
            pltpu.VMEM((2, Q, C), BF16),
            pltpu.VMEM((2, Q, C), BF16),
            pltpu.SemaphoreType.DMA((2, 4)),
            pltpu.SemaphoreType.DMA((2, 4)),
        ],
        compiler_params=pltpu.CompilerParams(collective_id=0),
    )(A, B)


# device time: 28546 ns/iter; 1.1634x vs baseline; 1.1405x over previous
import jax
import jax.numpy as jnp
from jax import lax
from jax.experimental import pallas as pl
from jax.experimental.pallas import tpu as pltpu

N_DEV = 4
M = 1024
N = 1024
H = M // 2
Q = M // 4
CH = N // 4

F32 = jnp.float32
BF16 = jnp.bfloat16


def _gelu(z):
    return 0.5 * z * (1.0 + jnp.tanh(0.7978845608 * (z + 0.044715 * z * z * z)))


def kernel(A, B):
    def body(
        a_ref,
        b_ref,
        out_ref,
        h_send,
        h_recv,
        q_send,
        q_recv,
        send_sems,
        recv_sems,
    ):
        d = lax.axis_index("i")
        p1 = d ^ 1
        p2 = 3 - d

        keep0 = (d ^ (d >> 1)) & 1
        qi0 = keep0 * 2 + (d >> 1)
        keep1 = d >> 1
        qi1 = keep1 * 2 + (d & 1)

        lanes = []
        for li, (g, c) in enumerate([(0, 0), (1, 0), (0, 1), (1, 1)]):
            keep = keep0 if g == 0 else keep1
            qi = qi0 if g == 0 else qi1
            qo = keep * 2 + (1 - (qi - keep * 2))
            lanes.append(
                dict(
                    li=li,
                    pa=p1 if g == 0 else p2,
                    pb=p2 if g == 0 else p1,
                    keep_r=keep * H,
                    send_r=(1 - keep) * H,
                    qi_r=qi * Q,
                    off_qi=(qi - keep * 2) * Q,
                    off_qo=(1 - (qi - keep * 2)) * Q,
                    col=g * (2 * CH) + c * CH,
                )
            )

        barrier_sem = pltpu.get_barrier_semaphore()
        for nbr in [p1, p2]:
            pl.semaphore_signal(
                barrier_sem,
                inc=1,
                device_id=(nbr,),
                device_id_type=pl.DeviceIdType.MESH,
            )
        pl.semaphore_wait(barrier_sem, 2)

        def mm(r, nrows, c):
            a = a_ref[pl.ds(r, nrows), :].astype(BF16)
            b = b_ref[:, pl.ds(c, CH)].astype(BF16)
            return jnp.dot(a, b, preferred_element_type=F32)

        rdma1 = []
        for ln in lanes:
            li = ln["li"]
            h_send[li] = mm(ln["send_r"], H, ln["col"]).astype(BF16)
            r = pltpu.make_async_remote_copy(
                src_ref=h_send.at[li],
                dst_ref=h_recv.at[li],
                send_sem=send_sems.at[li, 0],
                recv_sem=recv_sems.at[li, 0],
                device_id=(ln["pa"],),
                device_id_type=pl.DeviceIdType.MESH,
            )
            r.start()
            rdma1.append(r)
        mm_qo = [mm(ln["keep_r"] + ln["off_qo"], Q, ln["col"]) for ln in lanes]
        mm_qi = [mm(ln["qi_r"], Q, ln["col"]) for ln in lanes]

        rdma2 = []
        for ln, r1 in zip(lanes, rdma1):
            r1.wait()
            li = ln["li"]
            q_send[li] = (
                mm_qo[li] + h_recv[li, pl.ds(ln["off_qo"], Q), :].astype(F32)
            ).astype(BF16)
            r = pltpu.make_async_remote_copy(
                src_ref=q_send.at[li],
                dst_ref=q_recv.at[li],
                send_sem=send_sems.at[li, 1],
                recv_sem=recv_sems.at[li, 1],
                device_id=(ln["pb"],),
                device_id_type=pl.DeviceIdType.MESH,
            )
            r.start()
            rdma2.append(r)

        rdma3 = []
        for ln, r2 in zip(lanes, rdma2):
            li = ln["li"]
            zqp = mm_qi[li] + h_recv[li, pl.ds(ln["off_qi"], Q), :].astype(F32)
            r2.wait()
            gq = _gelu(zqp + q_recv[li].astype(F32))
            qs = (pl.ds(ln["qi_r"], Q), pl.ds(ln["col"], CH))
            out_ref[qs] = gq.astype(BF16)
            r = pltpu.make_async_remote_copy(
                src_ref=out_ref.at[qs],
                dst_ref=out_ref.at[qs],
                send_sem=send_sems.at[li, 2],
                recv_sem=recv_sems.at[li, 2],
                device_id=(ln["pb"],),
                device_id_type=pl.DeviceIdType.MESH,
            )
            r.start()
            rdma3.append(r)

        rdma4 = []
        for ln, r3 in zip(lanes, rdma3):
            r3.wait()
            li = ln["li"]
            hs = (pl.ds(ln["keep_r"], H), pl.ds(ln["col"], CH))
            r = pltpu.make_async_remote_copy(
                src_ref=out_ref.at[hs],
                dst_ref=out_ref.at[hs],
                send_sem=send_sems.at[li, 3],
                recv_sem=recv_sems.at[li, 3],
                device_id=(ln["pa"],),
                device_id_type=pl.DeviceIdType.MESH,
            )
            r.start()
            rdma4.append(r)
        for r4 in rdma4:
            r4.wait()

    return pl.pallas_call(
        body,
        out_shape=jax.ShapeDtypeStruct((M, N), BF16),
        in_specs=[
            pl.BlockSpec(memory_space=pltpu.VMEM),
            pl.BlockSpec(memory_space=pltpu.VMEM),
        ],
        out_specs=pl.BlockSpec(memory_space=pltpu.VMEM),
        scratch_shapes=[
            pltpu.VMEM((4, H, CH), BF16),
            pltpu.VMEM((4, H, CH), BF16),
            pltpu.VMEM((4, Q, CH), BF16),
            pltpu.VMEM((4, Q, CH), BF16),
            pltpu.SemaphoreType.DMA((4, 4)),
            pltpu.SemaphoreType.DMA((4, 4)),
        ],
        compiler_params=pltpu.CompilerParams(collective_id=0),
    )(A, B)


# device time: 27577 ns/iter; 1.2043x vs baseline; 1.0351x over previous
import jax
import jax.numpy as jnp
from jax import lax
from jax.experimental import pallas as pl
from jax.experimental.pallas import tpu as pltpu

N_DEV = 4
M = 1024
N = 1024
H = M // 2
Q = M // 4
NSUB = 4
NL = 2 * NSUB
CH = N // NL

F32 = jnp.float32
BF16 = jnp.bfloat16


def _gelu(z):
    return 0.5 * z * (1.0 + jnp.tanh(0.7978845608 * (z + 0.044715 * z * z * z)))


def kernel(A, B):
    def body(
        a_ref,
        b_ref,
        out_ref,
        h_send,
        h_recv,
        q_send,
        q_recv,
        send_sems,
        recv_sems,
    ):
        d = lax.axis_index("i")
        p1 = d ^ 1
        p2 = 3 - d

        keep0 = (d ^ (d >> 1)) & 1
        qi0 = keep0 * 2 + (d >> 1)
        keep1 = d >> 1
        qi1 = keep1 * 2 + (d & 1)

        lanes = []
        gc = [(g, c) for c in range(NSUB) for g in (0, 1)]
        for li, (g, c) in enumerate(gc):
            keep = keep0 if g == 0 else keep1
            qi = qi0 if g == 0 else qi1
            qo = keep * 2 + (1 - (qi - keep * 2))
            lanes.append(
                dict(
                    li=li,
                    pa=p1 if g == 0 else p2,
                    pb=p2 if g == 0 else p1,
                    keep_r=keep * H,
                    send_r=(1 - keep) * H,
                    qi_r=qi * Q,
                    off_qi=(qi - keep * 2) * Q,
                    off_qo=(1 - (qi - keep * 2)) * Q,
                    col=g * (NSUB * CH) + c * CH,
                )
            )

        barrier_sem = pltpu.get_barrier_semaphore()
        for nbr in [p1, p2]:
            pl.semaphore_signal(
                barrier_sem,
                inc=1,
                device_id=(nbr,),
                device_id_type=pl.DeviceIdType.MESH,
            )
        pl.semaphore_wait(barrier_sem, 2)

        def mm(r, nrows, c):
            a = a_ref[pl.ds(r, nrows), :].astype(BF16)
            b = b_ref[:, pl.ds(c, CH)].astype(BF16)
            return jnp.dot(a, b, preferred_element_type=F32)

        rdma1 = []
        for ln in lanes:
            li = ln["li"]
            h_send[li] = mm(ln["send_r"], H, ln["col"]).astype(BF16)
            r = pltpu.make_async_remote_copy(
                src_ref=h_send.at[li],
                dst_ref=h_recv.at[li],
                send_sem=send_sems.at[li, 0],
                recv_sem=recv_sems.at[li, 0],
                device_id=(ln["pa"],),
                device_id_type=pl.DeviceIdType.MESH,
            )
            r.start()
            rdma1.append(r)
        mm_qo = [mm(ln["keep_r"] + ln["off_qo"], Q, ln["col"]) for ln in lanes]
        mm_qi = [mm(ln["qi_r"], Q, ln["col"]) for ln in lanes]

        rdma2 = []
        for ln, r1 in zip(lanes, rdma1):
            r1.wait()
            li = ln["li"]
            q_send[li] = (
                mm_qo[li] + h_recv[li, pl.ds(ln["off_qo"], Q), :].astype(F32)
            ).astype(BF16)
            r = pltpu.make_async_remote_copy(
                src_ref=q_send.at[li],
                dst_ref=q_recv.at[li],
                send_sem=send_sems.at[li, 1],
                recv_sem=recv_sems.at[li, 1],
                device_id=(ln["pb"],),
                device_id_type=pl.DeviceIdType.MESH,
            )
            r.start()
            rdma2.append(r)

        rdma3 = []
        for ln, r2 in zip(lanes, rdma2):
            li = ln["li"]
            zqp = mm_qi[li] + h_recv[li, pl.ds(ln["off_qi"], Q), :].astype(F32)
            r2.wait()
            gq = _gelu(zqp + q_recv[li].astype(F32))
            qs = (pl.ds(ln["qi_r"], Q), pl.ds(ln["col"], CH))
            out_ref[qs] = gq.astype(BF16)
            r = pltpu.make_async_remote_copy(
                src_ref=out_ref.at[qs],
                dst_ref=out_ref.at[qs],
                send_sem=send_sems.at[li, 2],
                recv_sem=recv_sems.at[li, 2],
                device_id=(ln["pb"],),
                device_id_type=pl.DeviceIdType.MESH,
            )
            r.start()
            rdma3.append(r)

        rdma4 = []
        for ln, r3 in zip(lanes, rdma3):
            r3.wait()
            li = ln["li"]
            hs = (pl.ds(ln["keep_r"], H), pl.ds(ln["col"], CH))
            r = pltpu.make_async_remote_copy(
                src_ref=out_ref.at[hs],
                dst_ref=out_ref.at[hs],
                send_sem=send_sems.at[li, 3],
                recv_sem=recv_sems.at[li, 3],
                device_id=(ln["pa"],),
                device_id_type=pl.DeviceIdType.MESH,
            )
            r.start()
            rdma4.append(r)
        for r4 in rdma4:
            r4.wait()

    return pl.pallas_call(
        body,
        out_shape=jax.ShapeDtypeStruct((M, N), BF16),
        in_specs=[
            pl.BlockSpec(memory_space=pltpu.VMEM),
            pl.BlockSpec(memory_space=pltpu.VMEM),
        ],
        out_specs=pl.BlockSpec(memory_space=pltpu.VMEM),
        scratch_shapes=[
            pltpu.VMEM((NL, H, CH), BF16),
            pltpu.VMEM((NL, H, CH), BF16),
            pltpu.VMEM((NL, Q, CH), BF16),
            pltpu.VMEM((NL, Q, CH), BF16),
            pltpu.SemaphoreType.DMA((NL, 4)),
            pltpu.SemaphoreType.DMA((NL, 4)),
        ],
        compiler_params=pltpu.CompilerParams(collective_id=0),
    )(A, B)
